# baseline (speedup 1.0000x reference)
import functools
import math

import jax
import jax.numpy as jnp
from jax import lax
from jax.experimental import pallas as pl
from jax.experimental.pallas import tpu as pltpu
from jax.experimental.pallas import tpu_sc as plsc

N, D, FF, E, K = 2048, 1024, 4096, 8, 2
CAP = int(math.ceil(N * 1.05 * K / E))
CPAD = 576
ROWS = E * CPAD
HEXP = E // 2
HROWS = ROWS // 2
MAPSZ = ROWS + 16
NC, NS = 2, 16
NW = NC * NS
HRPT = HROWS // NW
TPT = N // NW
CCH = 16
PB = 1024
DH = D // 2
FB = 1024
HB = FB // 2
NF = FF // FB


RB = 512
NRB = N // RB


def _pack_bf16(v):
    u = lax.bitcast_convert_type(v, jnp.int32)
    rnd = u + 0x7FFF + (lax.shift_right_logical(u, 16) & 1)
    b = lax.shift_right_logical(rnd, 16)
    half = v.shape[1] // 2
    return b[:, :half] | (b[:, half:] << 16)


def _router_body(x_ref, wg_ref, oh0_ref, oh1_ref, i0_ref, i1_ref,
                 g0_ref, g1_ref, xb_ref):
    xb_ref[...] = _pack_bf16(x_ref[...])
    logits = jnp.dot(x_ref[...], wg_ref[...],
                     preferred_element_type=jnp.float32)
    m = jnp.max(logits, axis=1, keepdims=True)
    u = jnp.exp(logits - m)
    probs = u / jnp.sum(u, axis=1, keepdims=True)
    ecol = lax.broadcasted_iota(jnp.int32, (RB, E), 1)
    v0 = jnp.max(probs, axis=1, keepdims=True)
    i0 = jnp.min(jnp.where(probs == v0, ecol, E), axis=1, keepdims=True)
    pm = jnp.where(ecol == i0, -1.0, probs)
    v1 = jnp.max(pm, axis=1, keepdims=True)
    i1 = jnp.min(jnp.where(pm == v1, ecol, E), axis=1, keepdims=True)
    den = v0 + v1 + 1e-9
    oh0_ref[...] = (ecol == i0).astype(jnp.float32)
    oh1_ref[...] = (ecol == i1).astype(jnp.float32)
    i0_ref[...] = i0
    i1_ref[...] = i1
    g0_ref[...] = v0 / den
    g1_ref[...] = v1 / den


_router = pl.pallas_call(
    _router_body,
    grid=(NRB,),
    in_specs=[
        pl.BlockSpec((RB, D), lambda i: (i, 0)),
        pl.BlockSpec((D, E), lambda i: (0, 0)),
    ],
    out_specs=(
        pl.BlockSpec((RB, E), lambda i: (i, 0)),
        pl.BlockSpec((RB, E), lambda i: (i, 0)),
        pl.BlockSpec((RB, 1), lambda i: (i, 0)),
        pl.BlockSpec((RB, 1), lambda i: (i, 0)),
        pl.BlockSpec((RB, 1), lambda i: (i, 0)),
        pl.BlockSpec((RB, 1), lambda i: (i, 0)),
        pl.BlockSpec((RB, DH), lambda i: (i, 0)),
    ),
    out_shape=(
        jax.ShapeDtypeStruct((N, E), jnp.float32),
        jax.ShapeDtypeStruct((N, E), jnp.float32),
        jax.ShapeDtypeStruct((N, 1), jnp.int32),
        jax.ShapeDtypeStruct((N, 1), jnp.int32),
        jax.ShapeDtypeStruct((N, 1), jnp.float32),
        jax.ShapeDtypeStruct((N, 1), jnp.float32),
        jax.ShapeDtypeStruct((N, DH), jnp.int32),
    ),
)


def _pos_body(oh0_ref, oh1_ref, i0_ref, i1_ref, ga_ref, gb_ref,
              dest_ref, g_ref, run_ref, tri_ref):
    i = pl.program_id(0)
    nslot = N // PB

    @pl.when(i == 0)
    def _():
        run_ref[...] = jnp.zeros((1, E), jnp.float32)
        row = lax.broadcasted_iota(jnp.int32, (PB, PB), 0)
        col = lax.broadcasted_iota(jnp.int32, (PB, PB), 1)
        tri_ref[...] = (col < row).astype(jnp.float32)

    first = i < nslot
    oh = jnp.where(first, oh0_ref[...], oh1_ref[...])
    ic = jnp.where(first, i0_ref[...], i1_ref[...])
    ga = jnp.where(first, ga_ref[...], gb_ref[...])
    c = (jnp.dot(tri_ref[...], oh, preferred_element_type=jnp.float32)
         + run_ref[...])
    run_ref[...] = run_ref[...] + jnp.sum(oh, axis=0, keepdims=True)
    p = jnp.sum(c * oh, axis=1, keepdims=True).astype(jnp.int32)
    ok = p < CAP
    dest_ref[...] = jnp.where(ok, ic * CPAD + p, ROWS)
    g_ref[...] = jnp.where(ok, ga, 0.0)


_pos = pl.pallas_call(
    _pos_body,
    grid=((2 * N) // PB,),
    in_specs=[
        pl.BlockSpec((PB, E), lambda i: (i % (N // PB), 0)),
        pl.BlockSpec((PB, E), lambda i: (i % (N // PB), 0)),
        pl.BlockSpec((PB, 1), lambda i: (i % (N // PB), 0)),
        pl.BlockSpec((PB, 1), lambda i: (i % (N // PB), 0)),
        pl.BlockSpec((PB, 1), lambda i: (i % (N // PB), 0)),
        pl.BlockSpec((PB, 1), lambda i: (i % (N // PB), 0)),
    ],
    out_specs=(
        pl.BlockSpec((PB, 1), lambda i: (i, 0)),
        pl.BlockSpec((PB, 1), lambda i: (i, 0)),
    ),
    out_shape=(
        jax.ShapeDtypeStruct((2 * N, 1), jnp.int32),
        jax.ShapeDtypeStruct((2 * N, 1), jnp.float32),
    ),
    scratch_shapes=[pltpu.VMEM((1, E), jnp.float32),
                    pltpu.VMEM((PB, PB), jnp.float32)],
)


def _experts_body(xd_ref, w1_ref, b1_ref, w2_ref, b2_ref, y_ref, xds_ref):
    f = pl.program_id(1)

    @pl.when(f == 0)
    def _():
        y_ref[0] = jnp.broadcast_to(b2_ref[0], (CPAD, D))
        w = xd_ref[0]
        lo = lax.bitcast_convert_type(w << 16, jnp.float32)
        hi = lax.bitcast_convert_type(w & jnp.int32(-65536), jnp.float32)
        xds_ref[...] = jnp.concatenate([lo, hi], axis=1)

    xd = xds_ref[...]
    ha = jnp.dot(xd, w1_ref[0][:, :HB], preferred_element_type=jnp.float32)
    hb = jnp.dot(xd, w1_ref[0][:, HB:], preferred_element_type=jnp.float32)
    ga = jax.nn.gelu(ha + b1_ref[0][:, :HB])
    gb = jax.nn.gelu(hb + b1_ref[0][:, HB:])
    y_ref[0] = (y_ref[0]
                + jnp.dot(ga, w2_ref[0][:HB, :],
                          preferred_element_type=jnp.float32)
                + jnp.dot(gb, w2_ref[0][HB:, :],
                          preferred_element_type=jnp.float32))


_experts = pl.pallas_call(
    _experts_body,
    grid=(HEXP, NF),
    in_specs=[
        pl.BlockSpec((1, CPAD, DH), lambda e, f: (e, 0, 0)),
        pl.BlockSpec((1, D, FB), lambda e, f: (e, 0, f)),
        pl.BlockSpec((1, 1, FB), lambda e, f: (e, 0, f)),
        pl.BlockSpec((1, FB, D), lambda e, f: (e, f, 0)),
        pl.BlockSpec((1, 1, D), lambda e, f: (e, 0, 0)),
    ],
    out_specs=pl.BlockSpec((1, CPAD, D), lambda e, f: (e, 0, 0)),
    out_shape=jax.ShapeDtypeStruct((HEXP, CPAD, D), jnp.float32),
    scratch_shapes=[pltpu.VMEM((CPAD, D), jnp.float32)],
)


def _sc_mesh():
    return plsc.VectorSubcoreMesh(core_axis_name="c", subcore_axis_name="s",
                                  num_cores=NC, num_subcores=NS)


def _dispatch_call(x, destcat, h):
    @functools.partial(
        pl.kernel,
        mesh=_sc_mesh(),
        compiler_params=pltpu.CompilerParams(needs_layout_passes=False),
        out_type=jax.ShapeDtypeStruct((HROWS, DH), jnp.int32),
        scratch_types=[
            pltpu.VMEM((MAPSZ,), jnp.int32),
            pltpu.VMEM((2 * N,), jnp.int32),
            pltpu.VMEM((32, DH), jnp.int32),
            pltpu.VMEM((32, DH), jnp.int32),
            pltpu.VMEM((8, DH), jnp.int32),
        ] + [pltpu.SemaphoreType.DMA] * 6,
    )
    def disp(x_hbm, dest_hbm, out_hbm, map_v, dest_v, bufa, bufb, buft,
             sga, sgb, sgt, swa, swb, swt):
        pltpu.sync_copy(dest_hbm, dest_v)
        zeros = jnp.zeros((16,), jnp.int32)

        def zbody(i, c):
            map_v[pl.ds(i * 16, 16)] = zeros
            return c

        lax.fori_loop(0, MAPSZ // 16, zbody, 0)
        lane = lax.iota(jnp.int32, 16)

        def sbody(i, c):
            d = dest_v[pl.ds(i * 16, 16)]
            vals = lax.rem(i, N // 16) * 16 + lane
            plsc.store_scatter(map_v, [d], vals)
            return c

        lax.fori_loop(0, (2 * N) // 16, sbody, 0)
        wid = lax.axis_index("s") * NC + lax.axis_index("c")
        base = h * HROWS + wid * HRPT
        ob = wid * HRPT

        def gat(off, n, buf, sem):
            idx = map_v.at[pl.ds(base + off, n)]
            return pltpu.async_copy(x_hbm.at[idx], buf, sem)

        def put(off, n, buf, sem):
            return pltpu.async_copy(buf, out_hbm.at[pl.ds(ob + off, n)], sem)

        g0 = gat(0, 32, bufa, sga)
        g1 = gat(32, 32, bufb, sgb)
        g2 = gat(64, 8, buft, sgt)
        g0.wait()
        w0 = put(0, 32, bufa, swa)
        g1.wait()
        w1 = put(32, 32, bufb, swb)
        g2.wait()
        w2 = put(64, 8, buft, swt)
        w0.wait()
        w1.wait()
        w2.wait()

    return disp(x, destcat)


def _combine_call(y, destflat, gflat, h, partial=None):
    cch = CCH if partial is None else CCH // 2
    nch = TPT // cch
    scratch = [
        pltpu.VMEM((TPT,), jnp.int32),
        pltpu.VMEM((TPT,), jnp.int32),
        pltpu.VMEM((TPT,), jnp.float32),
        pltpu.VMEM((TPT,), jnp.float32),
        pltpu.VMEM((cch, D), jnp.float32),
        pltpu.VMEM((cch, D), jnp.float32),
        pltpu.VMEM((cch, D), jnp.float32),
        pltpu.VMEM((cch, D), jnp.float32),
        pltpu.VMEM((cch, D), jnp.float32),
        pltpu.VMEM((cch, D), jnp.float32),
    ] + [pltpu.SemaphoreType.DMA] * 6
    if partial is not None:
        scratch += ([pltpu.VMEM((cch, D), jnp.float32)] * 2
                    + [pltpu.SemaphoreType.DMA] * 2)

    @functools.partial(
        pl.kernel,
        mesh=_sc_mesh(),
        compiler_params=pltpu.CompilerParams(needs_layout_passes=False),
        out_type=jax.ShapeDtypeStruct((N, D), jnp.float32),
        scratch_types=scratch,
    )
    def comb(*refs):
        if partial is not None:
            (y_hbm, d_hbm, g_hbm, p_hbm, out_hbm,
             d0v, d1v, g0v, g1v, r0a, r1a, r0b, r1b, ov0, ov1,
             sg0a, sg1a, sg0b, sg1b, sw0, sw1, pb0, pb1, sp0, sp1) = refs
            pbs = [(pb0, sp0), (pb1, sp1)]
        else:
            (y_hbm, d_hbm, g_hbm, out_hbm,
             d0v, d1v, g0v, g1v, r0a, r1a, r0b, r1b, ov0, ov1,
             sg0a, sg1a, sg0b, sg1b, sw0, sw1) = refs
            pbs = None
        wid = lax.axis_index("s") * NC + lax.axis_index("c")
        tb = wid * TPT
        pltpu.sync_copy(d_hbm.at[pl.ds(tb, TPT)], d0v)
        pltpu.sync_copy(d_hbm.at[pl.ds(N + tb, TPT)], d1v)
        pltpu.sync_copy(g_hbm.at[pl.ds(tb, TPT)], g0v)
        pltpu.sync_copy(g_hbm.at[pl.ds(N + tb, TPT)], g1v)
        for j in range(TPT // 16):
            sl = pl.ds(j * 16, 16)
            d0 = d0v[sl]
            d1 = d1v[sl]
            if h == 0:
                g0v[sl] = jnp.where(d0 < HROWS, g0v[sl], 0.0)
                g1v[sl] = jnp.where(d1 < HROWS, g1v[sl], 0.0)
                d0v[sl] = jnp.minimum(d0, HROWS - 1)
                d1v[sl] = jnp.minimum(d1, HROWS - 1)
            else:
                g0v[sl] = jnp.where(d0 >= HROWS, g0v[sl], 0.0)
                g1v[sl] = jnp.where(d1 >= HROWS, g1v[sl], 0.0)
                d0v[sl] = jnp.minimum(jnp.maximum(d0 - HROWS, 0), HROWS - 1)
                d1v[sl] = jnp.minimum(jnp.maximum(d1 - HROWS, 0), HROWS - 1)
        sets = [(r0a, r1a, sg0a, sg1a), (r0b, r1b, sg0b, sg1b)]
        ovs = [(ov0, sw0), (ov1, sw1)]

        def gpair(k, s):
            r0, r1, s0, s1 = sets[s]
            sl = pl.ds(k * cch, cch)
            cps = [pltpu.async_copy(y_hbm.at[d0v.at[sl]], r0, s0),
                   pltpu.async_copy(y_hbm.at[d1v.at[sl]], r1, s1)]
            if pbs is not None:
                cps.append(pltpu.async_copy(
                    p_hbm.at[pl.ds(tb + k * cch, cch)], pbs[s][0], pbs[s][1]))
            return cps

        pend = gpair(0, 0)
        wpend = [None, None]
        for k in range(nch):
            s = k % 2
            r0, r1, _, _ = sets[s]
            for c in pend:
                c.wait()
            if k + 1 < nch:
                pend = gpair(k + 1, 1 - s)
            gbase = (k * cch) // 16 * 16
            goff = (k * cch) % 16
            ga16 = g0v[pl.ds(gbase, 16)]
            gb16 = g1v[pl.ds(gbase, 16)]
            ovb, swx = ovs[s]
            if wpend[s] is not None:
                wpend[s].wait()
            pb = pbs[s][0] if pbs is not None else None
            for i in range(cch):
                a = ga16[goff + i]
                b = gb16[goff + i]

                def vbody(j, c):
                    for q in range(8):
                        sl = pl.ds(j * 128 + q * 16, 16)
                        acc = a * r0[i, sl] + b * r1[i, sl]
                        if pb is not None:
                            acc = pb[i, sl] + acc
                        ovb[i, sl] = acc
                    return c

                lax.fori_loop(0, D // 128, vbody, 0)
            wpend[s] = pltpu.async_copy(
                ovb, out_hbm.at[pl.ds(tb + k * cch, cch)], swx)
        wpend[0].wait()
        wpend[1].wait()

    if partial is None:
        return comb(y, destflat, gflat)
    return comb(y, destflat, gflat, partial)


def kernel(x, Wg, W1, b1, W2, b2):
    oh0, oh1, i0, i1, ga, gb, _router_xb = _router(x, Wg)
    destcat, gcat = _pos(oh0, oh1, i0, i1, ga, gb)
    destflat = destcat.reshape(2 * N)
    gflat = gcat.reshape(2 * N)
    xduA = _dispatch_call(_router_xb, destflat, 0)
    xduB = _dispatch_call(_router_xb, destflat, 1)
    yA = _experts(xduA.reshape(HEXP, CPAD, DH), W1[:HEXP],
                  b1[:HEXP].reshape(HEXP, 1, FF), W2[:HEXP],
                  b2[:HEXP].reshape(HEXP, 1, D))
    yB = _experts(xduB.reshape(HEXP, CPAD, DH), W1[HEXP:],
                  b1[HEXP:].reshape(HEXP, 1, FF), W2[HEXP:],
                  b2[HEXP:].reshape(HEXP, 1, D))
    pA = _combine_call(yA.reshape(HROWS, D), destflat, gflat, 0)
    out = _combine_call(yB.reshape(HROWS, D), destflat, gflat, 1, partial=pA)
    return out


# speedup vs baseline: 1.4752x; 1.4752x over previous
import functools
import math

import jax
import jax.numpy as jnp
from jax import lax
from jax.experimental import pallas as pl
from jax.experimental.pallas import tpu as pltpu
from jax.experimental.pallas import tpu_sc as plsc

N, D, FF, E, K = 2048, 1024, 4096, 8, 2
CAP = int(math.ceil(N * 1.05 * K / E))
CPAD = 576
ROWS = E * CPAD
HEXP = E // 2
HROWS = ROWS // 2
MAPSZ = ROWS + 16
NC, NS = 2, 16
NW = NC * NS
HRPT = HROWS // NW
TPT = N // NW
CCH = 16
PB = 1024
DH = D // 2
FB = 1024
HB = FB // 2
NF = FF // FB


RB = 512
NRB = N // RB


def _pack_bf16(v):
    u = lax.bitcast_convert_type(v, jnp.int32)
    rnd = u + 0x7FFF + (lax.shift_right_logical(u, 16) & 1)
    b = lax.shift_right_logical(rnd, 16)
    half = v.shape[1] // 2
    return b[:, :half] | (b[:, half:] << 16)


def _router_body(x_ref, wg_ref, oh0_ref, oh1_ref, i0_ref, i1_ref,
                 g0_ref, g1_ref, xb_ref):
    xb_ref[...] = _pack_bf16(x_ref[...])
    logits = jnp.dot(x_ref[...], wg_ref[...],
                     preferred_element_type=jnp.float32)
    m = jnp.max(logits, axis=1, keepdims=True)
    u = jnp.exp(logits - m)
    probs = u / jnp.sum(u, axis=1, keepdims=True)
    ecol = lax.broadcasted_iota(jnp.int32, (RB, E), 1)
    v0 = jnp.max(probs, axis=1, keepdims=True)
    i0 = jnp.min(jnp.where(probs == v0, ecol, E), axis=1, keepdims=True)
    pm = jnp.where(ecol == i0, -1.0, probs)
    v1 = jnp.max(pm, axis=1, keepdims=True)
    i1 = jnp.min(jnp.where(pm == v1, ecol, E), axis=1, keepdims=True)
    den = v0 + v1 + 1e-9
    oh0_ref[...] = (ecol == i0).astype(jnp.float32)
    oh1_ref[...] = (ecol == i1).astype(jnp.float32)
    i0_ref[...] = i0
    i1_ref[...] = i1
    g0_ref[...] = v0 / den
    g1_ref[...] = v1 / den


_router = pl.pallas_call(
    _router_body,
    grid=(NRB,),
    in_specs=[
        pl.BlockSpec((RB, D), lambda i: (i, 0)),
        pl.BlockSpec((D, E), lambda i: (0, 0)),
    ],
    out_specs=(
        pl.BlockSpec((RB, E), lambda i: (i, 0)),
        pl.BlockSpec((RB, E), lambda i: (i, 0)),
        pl.BlockSpec((RB, 1), lambda i: (i, 0)),
        pl.BlockSpec((RB, 1), lambda i: (i, 0)),
        pl.BlockSpec((RB, 1), lambda i: (i, 0)),
        pl.BlockSpec((RB, 1), lambda i: (i, 0)),
        pl.BlockSpec((RB, DH), lambda i: (i, 0)),
    ),
    out_shape=(
        jax.ShapeDtypeStruct((N, E), jnp.float32),
        jax.ShapeDtypeStruct((N, E), jnp.float32),
        jax.ShapeDtypeStruct((N, 1), jnp.int32),
        jax.ShapeDtypeStruct((N, 1), jnp.int32),
        jax.ShapeDtypeStruct((N, 1), jnp.float32),
        jax.ShapeDtypeStruct((N, 1), jnp.float32),
        jax.ShapeDtypeStruct((N, DH), jnp.int32),
    ),
)


def _pos_body(oh0_ref, oh1_ref, i0_ref, i1_ref, ga_ref, gb_ref,
              dest_ref, g_ref, run_ref, tri_ref):
    i = pl.program_id(0)
    nslot = N // PB

    @pl.when(i == 0)
    def _():
        run_ref[...] = jnp.zeros((1, E), jnp.float32)
        row = lax.broadcasted_iota(jnp.int32, (PB, PB), 0)
        col = lax.broadcasted_iota(jnp.int32, (PB, PB), 1)
        tri_ref[...] = (col < row).astype(jnp.float32)

    first = i < nslot
    oh = jnp.where(first, oh0_ref[...], oh1_ref[...])
    ic = jnp.where(first, i0_ref[...], i1_ref[...])
    ga = jnp.where(first, ga_ref[...], gb_ref[...])
    c = (jnp.dot(tri_ref[...], oh, preferred_element_type=jnp.float32)
         + run_ref[...])
    run_ref[...] = run_ref[...] + jnp.sum(oh, axis=0, keepdims=True)
    p = jnp.sum(c * oh, axis=1, keepdims=True).astype(jnp.int32)
    ok = p < CAP
    dest_ref[...] = jnp.where(ok, ic * CPAD + p, ROWS)
    g_ref[...] = jnp.where(ok, ga, 0.0)


_pos = pl.pallas_call(
    _pos_body,
    grid=((2 * N) // PB,),
    in_specs=[
        pl.BlockSpec((PB, E), lambda i: (i % (N // PB), 0)),
        pl.BlockSpec((PB, E), lambda i: (i % (N // PB), 0)),
        pl.BlockSpec((PB, 1), lambda i: (i % (N // PB), 0)),
        pl.BlockSpec((PB, 1), lambda i: (i % (N // PB), 0)),
        pl.BlockSpec((PB, 1), lambda i: (i % (N // PB), 0)),
        pl.BlockSpec((PB, 1), lambda i: (i % (N // PB), 0)),
    ],
    out_specs=(
        pl.BlockSpec((PB, 1), lambda i: (i, 0)),
        pl.BlockSpec((PB, 1), lambda i: (i, 0)),
    ),
    out_shape=(
        jax.ShapeDtypeStruct((2 * N, 1), jnp.int32),
        jax.ShapeDtypeStruct((2 * N, 1), jnp.float32),
    ),
    scratch_shapes=[pltpu.VMEM((1, E), jnp.float32),
                    pltpu.VMEM((PB, PB), jnp.float32)],
)


def _experts_body(xd_ref, w1_ref, b1_ref, w2_ref, b2_ref, y_ref, xds_ref):
    f = pl.program_id(1)

    @pl.when(f == 0)
    def _():
        y_ref[0] = jnp.broadcast_to(b2_ref[0], (CPAD, D))
        w = xd_ref[0]
        lo = lax.bitcast_convert_type(w << 16, jnp.float32)
        hi = lax.bitcast_convert_type(w & jnp.int32(-65536), jnp.float32)
        xds_ref[...] = jnp.concatenate([lo, hi], axis=1)

    xd = xds_ref[...]
    ha = jnp.dot(xd, w1_ref[0][:, :HB], preferred_element_type=jnp.float32)
    hb = jnp.dot(xd, w1_ref[0][:, HB:], preferred_element_type=jnp.float32)
    ga = jax.nn.gelu(ha + b1_ref[0][:, :HB])
    gb = jax.nn.gelu(hb + b1_ref[0][:, HB:])
    y_ref[0] = (y_ref[0]
                + jnp.dot(ga, w2_ref[0][:HB, :],
                          preferred_element_type=jnp.float32)
                + jnp.dot(gb, w2_ref[0][HB:, :],
                          preferred_element_type=jnp.float32))


def _make_experts(h):
    eoff = h * HEXP
    return pl.pallas_call(
        _experts_body,
        grid=(HEXP, NF),
        in_specs=[
            pl.BlockSpec((1, CPAD, DH), lambda e, f: (e, 0, 0)),
            pl.BlockSpec((1, D, FB), lambda e, f: (eoff + e, 0, f)),
            pl.BlockSpec((1, 1, FB), lambda e, f: (eoff + e, 0, f)),
            pl.BlockSpec((1, FB, D), lambda e, f: (eoff + e, f, 0)),
            pl.BlockSpec((1, 1, D), lambda e, f: (eoff + e, 0, 0)),
        ],
        out_specs=pl.BlockSpec((1, CPAD, D), lambda e, f: (e, 0, 0)),
        out_shape=jax.ShapeDtypeStruct((HEXP, CPAD, D), jnp.float32),
        scratch_shapes=[pltpu.VMEM((CPAD, D), jnp.float32)],
    )


_experts_a = _make_experts(0)
_experts_b = _make_experts(1)


def _sc_mesh():
    return plsc.VectorSubcoreMesh(core_axis_name="c", subcore_axis_name="s",
                                  num_cores=NC, num_subcores=NS)


def _dispatch_call(x, destcat, h):
    @functools.partial(
        pl.kernel,
        mesh=_sc_mesh(),
        compiler_params=pltpu.CompilerParams(needs_layout_passes=False),
        out_type=jax.ShapeDtypeStruct((HROWS, DH), jnp.int32),
        scratch_types=[
            pltpu.VMEM((MAPSZ,), jnp.int32),
            pltpu.VMEM((2 * N,), jnp.int32),
            pltpu.VMEM((32, DH), jnp.int32),
            pltpu.VMEM((32, DH), jnp.int32),
            pltpu.VMEM((8, DH), jnp.int32),
        ] + [pltpu.SemaphoreType.DMA] * 6,
    )
    def disp(x_hbm, dest_hbm, out_hbm, map_v, dest_v, bufa, bufb, buft,
             sga, sgb, sgt, swa, swb, swt):
        pltpu.sync_copy(dest_hbm, dest_v)
        zeros = jnp.zeros((16,), jnp.int32)

        def zbody(i, c):
            map_v[pl.ds(i * 16, 16)] = zeros
            return c

        lax.fori_loop(0, MAPSZ // 16, zbody, 0)
        lane = lax.iota(jnp.int32, 16)

        def sbody(i, c):
            d = dest_v[pl.ds(i * 16, 16)]
            vals = lax.rem(i, N // 16) * 16 + lane
            plsc.store_scatter(map_v, [d], vals)
            return c

        lax.fori_loop(0, (2 * N) // 16, sbody, 0)
        wid = lax.axis_index("s") * NC + lax.axis_index("c")
        base = h * HROWS + wid * HRPT
        ob = wid * HRPT

        def gat(off, n, buf, sem):
            idx = map_v.at[pl.ds(base + off, n)]
            return pltpu.async_copy(x_hbm.at[idx], buf, sem)

        def put(off, n, buf, sem):
            return pltpu.async_copy(buf, out_hbm.at[pl.ds(ob + off, n)], sem)

        g0 = gat(0, 32, bufa, sga)
        g1 = gat(32, 32, bufb, sgb)
        g2 = gat(64, 8, buft, sgt)
        g0.wait()
        w0 = put(0, 32, bufa, swa)
        g1.wait()
        w1 = put(32, 32, bufb, swb)
        g2.wait()
        w2 = put(64, 8, buft, swt)
        w0.wait()
        w1.wait()
        w2.wait()

    return disp(x, destcat)


def _combine_call(y, destflat, gflat, h, partial=None):
    cch = CCH if partial is None else CCH // 2
    nch = TPT // cch
    scratch = [
        pltpu.VMEM((TPT,), jnp.int32),
        pltpu.VMEM((TPT,), jnp.int32),
        pltpu.VMEM((TPT,), jnp.float32),
        pltpu.VMEM((TPT,), jnp.float32),
        pltpu.VMEM((cch, D), jnp.float32),
        pltpu.VMEM((cch, D), jnp.float32),
        pltpu.VMEM((cch, D), jnp.float32),
        pltpu.VMEM((cch, D), jnp.float32),
        pltpu.VMEM((cch, D), jnp.float32),
        pltpu.VMEM((cch, D), jnp.float32),
    ] + [pltpu.SemaphoreType.DMA] * 6
    if partial is not None:
        scratch += ([pltpu.VMEM((cch, D), jnp.float32)] * 2
                    + [pltpu.SemaphoreType.DMA] * 2)

    @functools.partial(
        pl.kernel,
        mesh=_sc_mesh(),
        compiler_params=pltpu.CompilerParams(needs_layout_passes=False),
        out_type=jax.ShapeDtypeStruct((N, D), jnp.float32),
        scratch_types=scratch,
    )
    def comb(*refs):
        if partial is not None:
            (y_hbm, d_hbm, g_hbm, p_hbm, out_hbm,
             d0v, d1v, g0v, g1v, r0a, r1a, r0b, r1b, ov0, ov1,
             sg0a, sg1a, sg0b, sg1b, sw0, sw1, pb0, pb1, sp0, sp1) = refs
            pbs = [(pb0, sp0), (pb1, sp1)]
        else:
            (y_hbm, d_hbm, g_hbm, out_hbm,
             d0v, d1v, g0v, g1v, r0a, r1a, r0b, r1b, ov0, ov1,
             sg0a, sg1a, sg0b, sg1b, sw0, sw1) = refs
            pbs = None
        wid = lax.axis_index("s") * NC + lax.axis_index("c")
        tb = wid * TPT
        pltpu.sync_copy(d_hbm.at[pl.ds(tb, TPT)], d0v)
        pltpu.sync_copy(d_hbm.at[pl.ds(N + tb, TPT)], d1v)
        pltpu.sync_copy(g_hbm.at[pl.ds(tb, TPT)], g0v)
        pltpu.sync_copy(g_hbm.at[pl.ds(N + tb, TPT)], g1v)
        for j in range(TPT // 16):
            sl = pl.ds(j * 16, 16)
            d0 = d0v[sl]
            d1 = d1v[sl]
            if h == 0:
                g0v[sl] = jnp.where(d0 < HROWS, g0v[sl], 0.0)
                g1v[sl] = jnp.where(d1 < HROWS, g1v[sl], 0.0)
                d0v[sl] = jnp.minimum(d0, HROWS - 1)
                d1v[sl] = jnp.minimum(d1, HROWS - 1)
            else:
                g0v[sl] = jnp.where(d0 >= HROWS, g0v[sl], 0.0)
                g1v[sl] = jnp.where(d1 >= HROWS, g1v[sl], 0.0)
                d0v[sl] = jnp.minimum(jnp.maximum(d0 - HROWS, 0), HROWS - 1)
                d1v[sl] = jnp.minimum(jnp.maximum(d1 - HROWS, 0), HROWS - 1)
        sets = [(r0a, r1a, sg0a, sg1a), (r0b, r1b, sg0b, sg1b)]
        ovs = [(ov0, sw0), (ov1, sw1)]

        def gpair(k, s):
            r0, r1, s0, s1 = sets[s]
            sl = pl.ds(k * cch, cch)
            cps = [pltpu.async_copy(y_hbm.at[d0v.at[sl]], r0, s0),
                   pltpu.async_copy(y_hbm.at[d1v.at[sl]], r1, s1)]
            if pbs is not None:
                cps.append(pltpu.async_copy(
                    p_hbm.at[pl.ds(tb + k * cch, cch)], pbs[s][0], pbs[s][1]))
            return cps

        pend = gpair(0, 0)
        wpend = [None, None]
        for k in range(nch):
            s = k % 2
            r0, r1, _, _ = sets[s]
            for c in pend:
                c.wait()
            if k + 1 < nch:
                pend = gpair(k + 1, 1 - s)
            gbase = (k * cch) // 16 * 16
            goff = (k * cch) % 16
            ga16 = g0v[pl.ds(gbase, 16)]
            gb16 = g1v[pl.ds(gbase, 16)]
            ovb, swx = ovs[s]
            if wpend[s] is not None:
                wpend[s].wait()
            pb = pbs[s][0] if pbs is not None else None
            for i in range(cch):
                a = ga16[goff + i]
                b = gb16[goff + i]

                def vbody(j, c):
                    for q in range(8):
                        sl = pl.ds(j * 128 + q * 16, 16)
                        acc = a * r0[i, sl] + b * r1[i, sl]
                        if pb is not None:
                            acc = pb[i, sl] + acc
                        ovb[i, sl] = acc
                    return c

                lax.fori_loop(0, D // 128, vbody, 0)
            wpend[s] = pltpu.async_copy(
                ovb, out_hbm.at[pl.ds(tb + k * cch, cch)], swx)
        wpend[0].wait()
        wpend[1].wait()

    if partial is None:
        return comb(y, destflat, gflat)
    return comb(y, destflat, gflat, partial)


def kernel(x, Wg, W1, b1, W2, b2):
    oh0, oh1, i0, i1, ga, gb, _router_xb = _router(x, Wg)
    destcat, gcat = _pos(oh0, oh1, i0, i1, ga, gb)
    destflat = destcat.reshape(2 * N)
    gflat = gcat.reshape(2 * N)
    b1r = b1.reshape(E, 1, FF)
    b2r = b2.reshape(E, 1, D)
    xduA = _dispatch_call(_router_xb, destflat, 0)
    xduB = _dispatch_call(_router_xb, destflat, 1)
    yA = _experts_a(xduA.reshape(HEXP, CPAD, DH), W1, b1r, W2, b2r)
    yB = _experts_b(xduB.reshape(HEXP, CPAD, DH), W1, b1r, W2, b2r)
    pA = _combine_call(yA.reshape(HROWS, D), destflat, gflat, 0)
    out = _combine_call(yB.reshape(HROWS, D), destflat, gflat, 1, partial=pA)
    return out


# speedup vs baseline: 3.0697x; 2.0810x over previous
import functools
import math

import jax
import jax.numpy as jnp
from jax import lax
from jax.experimental import pallas as pl
from jax.experimental.pallas import tpu as pltpu
from jax.experimental.pallas import tpu_sc as plsc

N, D, FF, E, K = 2048, 1024, 4096, 8, 2
CAP = int(math.ceil(N * 1.05 * K / E))
CPAD = 544
ROWS = E * CPAD
MAPSZ = ROWS + 16
NC, NS = 2, 16
NW = NC * NS
RPT = ROWS // NW
TPT = N // NW
CCH = 16
PB = 1024
DH = D // 2
FB = 1024
HB = FB // 2
NF = FF // FB
RB = 512
NRB = N // RB


def _pack_bf16(v):
    u = lax.bitcast_convert_type(v, jnp.int32)
    rnd = u + 0x7FFF + (lax.shift_right_logical(u, 16) & 1)
    b = lax.shift_right_logical(rnd, 16)
    half = v.shape[1] // 2
    return b[:, :half] | (b[:, half:] << 16)


def _router_body(x_ref, wg_ref, oh0_ref, oh1_ref, i0_ref, i1_ref,
                 g0_ref, g1_ref, xb_ref):
    xb_ref[...] = _pack_bf16(x_ref[...])
    logits = jnp.dot(x_ref[...], wg_ref[...],
                     preferred_element_type=jnp.float32)
    m = jnp.max(logits, axis=1, keepdims=True)
    u = jnp.exp(logits - m)
    probs = u / jnp.sum(u, axis=1, keepdims=True)
    ecol = lax.broadcasted_iota(jnp.int32, (RB, E), 1)
    v0 = jnp.max(probs, axis=1, keepdims=True)
    i0 = jnp.min(jnp.where(probs == v0, ecol, E), axis=1, keepdims=True)
    pm = jnp.where(ecol == i0, -1.0, probs)
    v1 = jnp.max(pm, axis=1, keepdims=True)
    i1 = jnp.min(jnp.where(pm == v1, ecol, E), axis=1, keepdims=True)
    den = v0 + v1 + 1e-9
    oh0_ref[...] = (ecol == i0).astype(jnp.float32)
    oh1_ref[...] = (ecol == i1).astype(jnp.float32)
    i0_ref[...] = i0
    i1_ref[...] = i1
    g0_ref[...] = v0 / den
    g1_ref[...] = v1 / den


_router = pl.pallas_call(
    _router_body,
    grid=(NRB,),
    in_specs=[
        pl.BlockSpec((RB, D), lambda i: (i, 0)),
        pl.BlockSpec((D, E), lambda i: (0, 0)),
    ],
    out_specs=(
        pl.BlockSpec((RB, E), lambda i: (i, 0)),
        pl.BlockSpec((RB, E), lambda i: (i, 0)),
        pl.BlockSpec((RB, 1), lambda i: (i, 0)),
        pl.BlockSpec((RB, 1), lambda i: (i, 0)),
        pl.BlockSpec((RB, 1), lambda i: (i, 0)),
        pl.BlockSpec((RB, 1), lambda i: (i, 0)),
        pl.BlockSpec((RB, DH), lambda i: (i, 0)),
    ),
    out_shape=(
        jax.ShapeDtypeStruct((N, E), jnp.float32),
        jax.ShapeDtypeStruct((N, E), jnp.float32),
        jax.ShapeDtypeStruct((N, 1), jnp.int32),
        jax.ShapeDtypeStruct((N, 1), jnp.int32),
        jax.ShapeDtypeStruct((N, 1), jnp.float32),
        jax.ShapeDtypeStruct((N, 1), jnp.float32),
        jax.ShapeDtypeStruct((N, DH), jnp.int32),
    ),
)


def _pos_body(oh0_ref, oh1_ref, i0_ref, i1_ref, ga_ref, gb_ref,
              dest_ref, g_ref, run_ref, tri_ref):
    i = pl.program_id(0)
    nslot = N // PB

    @pl.when(i == 0)
    def _():
        run_ref[...] = jnp.zeros((1, E), jnp.float32)
        row = lax.broadcasted_iota(jnp.int32, (PB, PB), 0)
        col = lax.broadcasted_iota(jnp.int32, (PB, PB), 1)
        tri_ref[...] = (col < row).astype(jnp.float32)

    first = i < nslot
    oh = jnp.where(first, oh0_ref[...], oh1_ref[...])
    ic = jnp.where(first, i0_ref[...], i1_ref[...])
    ga = jnp.where(first, ga_ref[...], gb_ref[...])
    c = (jnp.dot(tri_ref[...], oh, preferred_element_type=jnp.float32)
         + run_ref[...])
    run_ref[...] = run_ref[...] + jnp.sum(oh, axis=0, keepdims=True)
    p = jnp.sum(c * oh, axis=1, keepdims=True).astype(jnp.int32)
    ok = p < CAP
    dest_ref[...] = jnp.where(ok, ic * CPAD + p, ROWS)
    g_ref[...] = jnp.where(ok, ga, 0.0)


_pos = pl.pallas_call(
    _pos_body,
    grid=((2 * N) // PB,),
    in_specs=[
        pl.BlockSpec((PB, E), lambda i: (i % (N // PB), 0)),
        pl.BlockSpec((PB, E), lambda i: (i % (N // PB), 0)),
        pl.BlockSpec((PB, 1), lambda i: (i % (N // PB), 0)),
        pl.BlockSpec((PB, 1), lambda i: (i % (N // PB), 0)),
        pl.BlockSpec((PB, 1), lambda i: (i % (N // PB), 0)),
        pl.BlockSpec((PB, 1), lambda i: (i % (N // PB), 0)),
    ],
    out_specs=(
        pl.BlockSpec((PB, 1), lambda i: (i, 0)),
        pl.BlockSpec((PB, 1), lambda i: (i, 0)),
    ),
    out_shape=(
        jax.ShapeDtypeStruct((2 * N, 1), jnp.int32),
        jax.ShapeDtypeStruct((2 * N, 1), jnp.float32),
    ),
    scratch_shapes=[pltpu.VMEM((1, E), jnp.float32),
                    pltpu.VMEM((PB, PB), jnp.float32)],
)


def _experts_body(xd_ref, w1_ref, b1_ref, w2_ref, b2_ref, y_ref, xds_ref):
    f = pl.program_id(1)

    @pl.when(f == 0)
    def _():
        y_ref[0] = jnp.broadcast_to(b2_ref[0], (CPAD, D))
        w = xd_ref[0]
        lo = lax.bitcast_convert_type(w << 16, jnp.float32)
        hi = lax.bitcast_convert_type(w & jnp.int32(-65536), jnp.float32)
        xds_ref[...] = jnp.concatenate([lo, hi], axis=1)

    xd = xds_ref[...]
    ha = jnp.dot(xd, w1_ref[0][:, :HB], preferred_element_type=jnp.float32)
    hb = jnp.dot(xd, w1_ref[0][:, HB:], preferred_element_type=jnp.float32)
    ga = jax.nn.gelu(ha + b1_ref[0][:, :HB])
    gb = jax.nn.gelu(hb + b1_ref[0][:, HB:])
    y_ref[0] = (y_ref[0]
                + jnp.dot(ga, w2_ref[0][:HB, :],
                          preferred_element_type=jnp.float32)
                + jnp.dot(gb, w2_ref[0][HB:, :],
                          preferred_element_type=jnp.float32))


_experts = pl.pallas_call(
    _experts_body,
    grid=(E, NF),
    in_specs=[
        pl.BlockSpec((1, CPAD, DH), lambda e, f: (e, 0, 0)),
        pl.BlockSpec((1, D, FB), lambda e, f: (e, 0, f)),
        pl.BlockSpec((1, 1, FB), lambda e, f: (e, 0, f)),
        pl.BlockSpec((1, FB, D), lambda e, f: (e, f, 0)),
        pl.BlockSpec((1, 1, D), lambda e, f: (e, 0, 0)),
    ],
    out_specs=pl.BlockSpec((1, CPAD, D), lambda e, f: (e, 0, 0)),
    out_shape=jax.ShapeDtypeStruct((E, CPAD, D), jnp.float32),
    scratch_shapes=[pltpu.VMEM((CPAD, D), jnp.float32)],
)


def _sc_mesh():
    return plsc.VectorSubcoreMesh(core_axis_name="c", subcore_axis_name="s",
                                  num_cores=NC, num_subcores=NS)


def _dispatch_call(x, destcat):
    @functools.partial(
        pl.kernel,
        mesh=_sc_mesh(),
        compiler_params=pltpu.CompilerParams(needs_layout_passes=False),
        out_type=jax.ShapeDtypeStruct((ROWS, DH), jnp.int32),
        scratch_types=[
            pltpu.VMEM((MAPSZ,), jnp.int32),
            pltpu.VMEM((2 * N,), jnp.int32),
            pltpu.VMEM((32, DH), jnp.int32),
            pltpu.VMEM((32, DH), jnp.int32),
            pltpu.VMEM((32, DH), jnp.int32),
            pltpu.VMEM((8, DH), jnp.int32),
        ] + [pltpu.SemaphoreType.DMA] * 8,
    )
    def disp(x_hbm, dest_hbm, out_hbm, map_v, dest_v, bufa, bufb, bufc, buft,
             sga, sgb, sgc, sgt, swa, swb, swc, swt):
        pltpu.sync_copy(dest_hbm, dest_v)
        zeros = jnp.zeros((16,), jnp.int32)

        def zbody(i, c):
            map_v[pl.ds(i * 16, 16)] = zeros
            return c

        lax.fori_loop(0, MAPSZ // 16, zbody, 0)
        lane = lax.iota(jnp.int32, 16)

        def sbody(i, c):
            d = dest_v[pl.ds(i * 16, 16)]
            vals = lax.rem(i, N // 16) * 16 + lane
            plsc.store_scatter(map_v, [d], vals)
            return c

        lax.fori_loop(0, (2 * N) // 16, sbody, 0)
        wid = lax.axis_index("s") * NC + lax.axis_index("c")
        base = wid * RPT

        def gat(off, n, buf, sem):
            idx = map_v.at[pl.ds(base + off, n)]
            return pltpu.async_copy(x_hbm.at[idx], buf, sem)

        def put(off, n, buf, sem):
            return pltpu.async_copy(buf, out_hbm.at[pl.ds(base + off, n)], sem)

        g0 = gat(0, 32, bufa, sga)
        g1 = gat(32, 32, bufb, sgb)
        g2 = gat(64, 32, bufc, sgc)
        g4 = gat(128, 8, buft, sgt)
        g0.wait()
        w0 = put(0, 32, bufa, swa)
        g1.wait()
        w1 = put(32, 32, bufb, swb)
        g2.wait()
        w2 = put(64, 32, bufc, swc)
        w0.wait()
        g3 = gat(96, 32, bufa, sga)
        g3.wait()
        w3 = put(96, 32, bufa, swa)
        g4.wait()
        w4 = put(128, 8, buft, swt)
        w1.wait()
        w2.wait()
        w3.wait()
        w4.wait()

    return disp(x, destcat)


def _combine_call(y, destflat, gflat):
    nch = TPT // CCH

    @functools.partial(
        pl.kernel,
        mesh=_sc_mesh(),
        out_type=jax.ShapeDtypeStruct((N, D), jnp.float32),
        scratch_types=[
            pltpu.VMEM((TPT,), jnp.int32),
            pltpu.VMEM((TPT,), jnp.int32),
            pltpu.VMEM((TPT,), jnp.float32),
            pltpu.VMEM((TPT,), jnp.float32),
            pltpu.VMEM((CCH, D), jnp.float32),
            pltpu.VMEM((CCH, D), jnp.float32),
            pltpu.VMEM((CCH, D), jnp.float32),
            pltpu.VMEM((CCH, D), jnp.float32),
            pltpu.VMEM((CCH, D), jnp.float32),
            pltpu.VMEM((CCH, D), jnp.float32),
        ] + [pltpu.SemaphoreType.DMA] * 6,
    )
    def comb(y_hbm, d_hbm, g_hbm, out_hbm,
             d0v, d1v, g0v, g1v, r0a, r1a, r0b, r1b, ov0, ov1,
             sg0a, sg1a, sg0b, sg1b, sw0, sw1):
        wid = lax.axis_index("s") * NC + lax.axis_index("c")
        tb = wid * TPT
        pltpu.sync_copy(d_hbm.at[pl.ds(tb, TPT)], d0v)
        pltpu.sync_copy(d_hbm.at[pl.ds(N + tb, TPT)], d1v)
        pltpu.sync_copy(g_hbm.at[pl.ds(tb, TPT)], g0v)
        pltpu.sync_copy(g_hbm.at[pl.ds(N + tb, TPT)], g1v)
        for j in range(TPT // 16):
            sl = pl.ds(j * 16, 16)
            d0v[sl] = jnp.minimum(d0v[sl], ROWS - 1)
            d1v[sl] = jnp.minimum(d1v[sl], ROWS - 1)
        sets = [(r0a, r1a, sg0a, sg1a), (r0b, r1b, sg0b, sg1b)]
        ovs = [(ov0, sw0), (ov1, sw1)]

        def gpair(k, s):
            r0, r1, s0, s1 = sets[s]
            sl = pl.ds(k * CCH, CCH)
            return (pltpu.async_copy(y_hbm.at[d0v.at[sl]], r0, s0),
                    pltpu.async_copy(y_hbm.at[d1v.at[sl]], r1, s1))

        pend = gpair(0, 0)
        wpend = [None, None]
        for k in range(nch):
            s = k % 2
            r0, r1, _, _ = sets[s]
            c0, c1 = pend
            c0.wait()
            c1.wait()
            if k + 1 < nch:
                pend = gpair(k + 1, 1 - s)
            ga16 = g0v[pl.ds(k * CCH, CCH)]
            gb16 = g1v[pl.ds(k * CCH, CCH)]
            ovb, swx = ovs[s]
            if wpend[s] is not None:
                wpend[s].wait()
            for i in range(CCH):
                a = ga16[i]
                b = gb16[i]

                def vbody(j, c):
                    for q in range(8):
                        sl = pl.ds(j * 128 + q * 16, 16)
                        ovb[i, sl] = a * r0[i, sl] + b * r1[i, sl]
                    return c

                lax.fori_loop(0, D // 128, vbody, 0)
            wpend[s] = pltpu.async_copy(
                ovb, out_hbm.at[pl.ds(tb + k * CCH, CCH)], swx)
        wpend[0].wait()
        wpend[1].wait()

    return comb(y, destflat, gflat)


def kernel(x, Wg, W1, b1, W2, b2):
    oh0, oh1, i0, i1, ga, gb, xb = _router(x, Wg)
    destcat, gcat = _pos(oh0, oh1, i0, i1, ga, gb)
    destflat = destcat.reshape(2 * N)
    gflat = gcat.reshape(2 * N)
    xdu = _dispatch_call(xb, destflat)
    y = _experts(xdu.reshape(E, CPAD, DH), W1, b1.reshape(E, 1, FF),
                 W2, b2.reshape(E, 1, D))
    out = _combine_call(y.reshape(ROWS, DH * 2), destflat, gflat)
    return out
